# baseline (speedup 1.0000x reference)
import functools

import jax
import jax.numpy as jnp
from jax import lax
from jax.experimental import pallas as pl
from jax.experimental.pallas import tpu as pltpu
from jax.experimental.pallas import tpu_sc as plsc

_NC = 2
_NS = 16
_NW = _NC * _NS
_LANES = 16

_GCH = 128
_CHUNK = 512


def _body(per_w, n_chunks, D, ids_hbm, off_hbm, table_hbm, out_hbm,
          idx_v, rows_v, off_v, sem):
    wid = lax.axis_index("s") * _NC + lax.axis_index("c")
    pltpu.sync_copy(off_hbm, off_v)
    offv = off_v[...]
    idx_rows = _CHUNK // _GCH
    base_idx_row = wid * (per_w // _GCH)
    base_out = wid * per_w

    def step(ci, carry):
        row0 = base_idx_row + ci * idx_rows
        pltpu.sync_copy(ids_hbm.at[pl.ds(row0, idx_rows)], idx_v)
        for r in range(idx_rows):
            for c in range(_GCH // _LANES):
                sl = pl.ds(c * _LANES, _LANES)
                idx_v[r, sl] = idx_v[r, sl] + offv
        copies = [
            pltpu.async_copy(table_hbm.at[idx_v.at[r]],
                             rows_v.at[pl.ds(r * _GCH, _GCH)], sem)
            for r in range(idx_rows)
        ]
        for cp in copies:
            cp.wait()
        pltpu.sync_copy(rows_v, out_hbm.at[pl.ds(base_out + ci * _CHUNK, _CHUNK)])
        return carry

    lax.fori_loop(0, n_chunks, step, 0)


def kernel(input_ids, table, offsets):
    B, L, H = input_ids.shape
    N, D = table.shape
    total = B * L * H
    per_w = total // _NW
    n_chunks = per_w // _CHUNK

    ids2 = input_ids.reshape(total // _GCH, _GCH)
    off16 = jnp.tile(offsets, _LANES // H).astype(jnp.int32)

    mesh = plsc.VectorSubcoreMesh(core_axis_name="c", subcore_axis_name="s")
    run = functools.partial(
        pl.kernel,
        out_type=jax.ShapeDtypeStruct((total, D), jnp.float32),
        mesh=mesh,
        compiler_params=pltpu.CompilerParams(use_tc_tiling_on_sc=False),
        scratch_types=[
            pltpu.VMEM((_CHUNK // _GCH, _GCH), jnp.int32),
            pltpu.VMEM((_CHUNK, D), jnp.float32),
            pltpu.VMEM((_LANES,), jnp.int32),
            pltpu.SemaphoreType.DMA,
        ],
    )(functools.partial(_body, per_w, n_chunks, D))
    out = run(ids2, off16, table)
    return out.reshape(B, L, H, D)


# speedup vs baseline: 1.1357x; 1.1357x over previous
import functools

import jax
import jax.numpy as jnp
from jax import lax
from jax.experimental import pallas as pl
from jax.experimental.pallas import tpu as pltpu
from jax.experimental.pallas import tpu_sc as plsc

_NC = 2
_NS = 16
_NW = _NC * _NS
_LANES = 16

_GCH = 128
_CHUNK = 256


def _body(per_w, n_chunks, D, ids_hbm, off_hbm, tablep_hbm, out_hbm,
          idx_v, rows_v, off_v, sem):
    wid = lax.axis_index("s") * _NC + lax.axis_index("c")
    pltpu.sync_copy(off_hbm, off_v)
    offv = off_v[...]
    idx_rows = _CHUNK // _GCH
    base_idx_row = wid * (per_w // _GCH)
    base_out = wid * per_w

    def step(ci, carry):
        row0 = base_idx_row + ci * idx_rows
        pltpu.sync_copy(ids_hbm.at[pl.ds(row0, idx_rows)], idx_v)
        for r in range(idx_rows):
            for c in range(_GCH // _LANES):
                sl = pl.ds(c * _LANES, _LANES)
                idx_v[r, sl] = idx_v[r, sl] + offv
        copies = [
            pltpu.async_copy(tablep_hbm.at[idx_v.at[r]],
                             rows_v.at[pl.ds(r * _GCH, _GCH)], sem)
            for r in range(idx_rows)
        ]
        for cp in copies:
            cp.wait()
        pltpu.sync_copy(rows_v,
                        out_hbm.at[pl.ds(base_out + ci * _CHUNK, _CHUNK)])
        return carry

    lax.fori_loop(0, n_chunks, step, 0)


def kernel(input_ids, table, offsets):
    B, L, H = input_ids.shape
    N, D = table.shape
    total = B * L * H
    per_w = total // _NW
    n_chunks = per_w // _CHUNK

    ids2 = input_ids.reshape(total // _GCH, _GCH)
    tablep = jnp.pad(table, ((0, 0), (0, D)))
    off16 = jnp.tile(offsets, _LANES // H).astype(jnp.int32)

    mesh = plsc.VectorSubcoreMesh(core_axis_name="c", subcore_axis_name="s")
    run = functools.partial(
        pl.kernel,
        out_type=jax.ShapeDtypeStruct((total, 2 * D), jnp.float32),
        mesh=mesh,
        compiler_params=pltpu.CompilerParams(use_tc_tiling_on_sc=True),
        scratch_types=[
            pltpu.VMEM((_CHUNK // _GCH, _GCH), jnp.int32),
            pltpu.VMEM((_CHUNK, 2 * D), jnp.float32),
            pltpu.VMEM((_LANES,), jnp.int32),
            pltpu.SemaphoreType.DMA,
        ],
    )(functools.partial(_body, per_w, n_chunks, D))
    out = run(ids2, off16, tablep)
    return out[:, :D].reshape(B, L, H, D)


# speedup vs baseline: 1.1928x; 1.0502x over previous
import functools

import jax
import jax.numpy as jnp
from jax import lax
from jax.experimental import pallas as pl
from jax.experimental.pallas import tpu as pltpu
from jax.experimental.pallas import tpu_sc as plsc

_NC = 2
_NS = 16
_NW = _NC * _NS
_LANES = 16

_GCH = 128
_CHUNK = 256
_NBUF = 2


def _body(per_w, n_chunks, ids_hbm, off_hbm, tablep_hbm, out_hbm,
          idx_bufs, row_bufs, off_v, gsems, osems):
    wid = lax.axis_index("s") * _NC + lax.axis_index("c")
    pltpu.sync_copy(off_hbm, off_v)
    offv = off_v[...]
    idx_rows = _CHUNK // _GCH
    base_idx_row = wid * (per_w // _GCH)
    base_out = wid * per_w

    def load_and_fire(ci, b):
        idx_v, rows_v = idx_bufs[b], row_bufs[b]
        pltpu.sync_copy(ids_hbm.at[pl.ds(base_idx_row + ci * idx_rows, idx_rows)],
                        idx_v)
        for r in range(idx_rows):
            for c in range(_GCH // _LANES):
                sl = pl.ds(c * _LANES, _LANES)
                idx_v[r, sl] = idx_v[r, sl] + offv
        return [
            pltpu.async_copy(tablep_hbm.at[idx_v.at[r]],
                             rows_v.at[pl.ds(r * _GCH, _GCH)], gsems[b])
            for r in range(idx_rows)
        ]

    gcopies = {0: load_and_fire(0, 0)}
    ocopies = {}
    for ci in range(n_chunks):
        b = ci % _NBUF
        for cp in gcopies.pop(ci):
            cp.wait()
        ocopies[ci] = pltpu.async_copy(
            row_bufs[b], out_hbm.at[pl.ds(base_out + ci * _CHUNK, _CHUNK)],
            osems[b])
        if ci + 1 < n_chunks:
            nb = (ci + 1) % _NBUF
            if ci >= 1:
                ocopies.pop(ci - 1).wait()
            gcopies[ci + 1] = load_and_fire(ci + 1, nb)
    for ci in list(ocopies):
        ocopies.pop(ci).wait()


def kernel(input_ids, table, offsets):
    B, L, H = input_ids.shape
    N, D = table.shape
    total = B * L * H
    per_w = total // _NW
    n_chunks = per_w // _CHUNK

    ids2 = input_ids.reshape(total // _GCH, _GCH)
    tablep = jnp.pad(table, ((0, 0), (0, D)))
    off16 = jnp.tile(offsets, _LANES // H).astype(jnp.int32)

    mesh = plsc.VectorSubcoreMesh(core_axis_name="c", subcore_axis_name="s")
    run = functools.partial(
        pl.kernel,
        out_type=jax.ShapeDtypeStruct((total, 2 * D), jnp.float32),
        mesh=mesh,
        compiler_params=pltpu.CompilerParams(use_tc_tiling_on_sc=True),
        scratch_types=[
            [pltpu.VMEM((_CHUNK // _GCH, _GCH), jnp.int32) for _ in range(_NBUF)],
            [pltpu.VMEM((_CHUNK, 2 * D), jnp.float32) for _ in range(_NBUF)],
            pltpu.VMEM((_LANES,), jnp.int32),
            [pltpu.SemaphoreType.DMA for _ in range(_NBUF)],
            [pltpu.SemaphoreType.DMA for _ in range(_NBUF)],
        ],
    )(functools.partial(_body, per_w, n_chunks))
    out = run(ids2, off16, tablep)
    return out[:, :D].reshape(B, L, H, D)
